# Initial kernel scaffold; baseline (speedup 1.0000x reference)
import jax
import jax.numpy as jnp
from jax.experimental import pallas as pl


def kernel(features, node_order, adjacency_list, edge_order, params):
    raise NotImplementedError("write your pallas kernel here")


# speedup vs baseline: 8.7733x; 8.7733x over previous
from functools import partial

import jax
import jax.numpy as jnp
from jax.experimental import pallas as pl
from jax.experimental.pallas import tpu as pltpu

_CHUNK = 2048


def _level_sizes(n):
    sizes = []
    d = 0
    while (1 << d) - 1 < n:
        s = (1 << d) - 1
        sizes.append(min((1 << (d + 1)) - 1, n) - s)
        d += 1
    return sizes


def _mlp_body(x_ref, w1_ref, b1_ref, w2_ref, b2_ref, w3_ref, b3_ref,
              w4_ref, b4_ref, wiou_ref, biou_ref, wf_ref, bf_ref, out_ref):
    def dot(a, b):
        return jax.lax.dot_general(a, b, (((1,), (0,)), ((), ())),
                                   preferred_element_type=jnp.float32)

    x = x_ref[...]
    for w_ref, b_ref in ((w1_ref, b1_ref), (w2_ref, b2_ref),
                         (w3_ref, b3_ref), (w4_ref, b4_ref)):
        x = jnp.maximum(dot(x, w_ref[...]) + b_ref[...], 0.0)
    xiou = dot(x, wiou_ref[...]) + biou_ref[...]
    xf = dot(x, wf_ref[...]) + bf_ref[...]
    out_ref[...] = jnp.concatenate([xiou, xf], axis=1)


def _sweep_body(sizes, hdim, xfiou_ref, uiou_ref, uf_ref, hc_ref,
                bufa_ref, bufb_ref, xs_ref, sem_in, sem_out):
    uiou = uiou_ref[...]
    uf = uf_ref[...]
    nd_last = sizes[-1]
    za = (nd_last // 8) * 8
    bufa_rows = bufa_ref.shape[0]
    bufa_ref[za:bufa_rows, :] = jnp.zeros(
        (bufa_rows - za, bufa_ref.shape[1]), jnp.float32)

    def level(d, buf_ref, obuf_ref, seg_full):
        n_d = sizes[d]
        s_g = (1 << d) - 1
        cp_in = pltpu.make_async_copy(
            xfiou_ref.at[pl.ds(s_g, n_d)], xs_ref.at[pl.ds(0, n_d)], sem_in)
        cp_in.start()
        cp_in.wait()
        for a in range(0, n_d, _CHUNK):
            b = min(a + _CHUNK, n_d)
            xio = xs_ref[a:b, :]
            xiou = xio[:, 0:3 * hdim]
            xf = xio[:, 3 * hdim:4 * hdim]
            if a < seg_full:
                che = obuf_ref[2 * a:2 * b:2, :]
                cho = obuf_ref[2 * a + 1:2 * b:2, :]
                he = che[:, 0:hdim]
                ce = che[:, hdim:2 * hdim]
                ho = cho[:, 0:hdim]
                co = cho[:, hdim:2 * hdim]
                iou = xiou + jax.lax.dot_general(
                    he + ho, uiou, (((1,), (0,)), ((), ())),
                    preferred_element_type=jnp.float32)
                f_e = jax.nn.sigmoid(xf + jax.lax.dot_general(
                    he, uf, (((1,), (0,)), ((), ())),
                    preferred_element_type=jnp.float32))
                f_o = jax.nn.sigmoid(xf + jax.lax.dot_general(
                    ho, uf, (((1,), (0,)), ((), ())),
                    preferred_element_type=jnp.float32))
                csum = f_e * ce + f_o * co
            else:
                iou = xiou
                csum = None
            i_g = jax.nn.sigmoid(iou[:, 0:hdim])
            o_g = jax.nn.sigmoid(iou[:, hdim:2 * hdim])
            u_g = jnp.tanh(iou[:, 2 * hdim:3 * hdim])
            c_new = i_g * u_g
            if csum is not None:
                c_new = c_new + csum
            h_new = o_g * jnp.tanh(c_new)
            buf_ref[a:b, :] = jnp.concatenate([h_new, c_new], axis=1)
        cp_out = pltpu.make_async_copy(
            buf_ref.at[pl.ds(0, n_d)], hc_ref.at[pl.ds(s_g, n_d)], sem_out)
        cp_out.start()
        cp_out.wait()

    nlev = len(sizes)
    for d in range(nlev - 1, -1, -1):
        buf_ref = bufa_ref if (nlev - 1 - d) % 2 == 0 else bufb_ref
        obuf_ref = bufb_ref if (nlev - 1 - d) % 2 == 0 else bufa_ref
        if d == nlev - 1:
            seg_full = 0
        elif d == nlev - 2:
            seg_full = min(bufa_ref.shape[0] // 2, sizes[d])
            seg_full = ((seg_full + _CHUNK - 1) // _CHUNK) * _CHUNK
            seg_full = min(seg_full, sizes[d])
        else:
            seg_full = sizes[d]
        level(d, buf_ref, obuf_ref, seg_full)


def _cls_body(hc_ref, w_ref, b_ref, out_ref):
    h = hc_ref[:, 0:w_ref.shape[0]]
    logits = jax.lax.dot_general(h, w_ref[...], (((1,), (0,)), ((), ())),
                                 preferred_element_type=jnp.float32)
    logits = logits + b_ref[...]
    m = jnp.max(logits, axis=1, keepdims=True)
    ex = jnp.exp(logits - m)
    out_ref[...] = ex / jnp.sum(ex, axis=1, keepdims=True)


def kernel(features, node_order, adjacency_list, edge_order, params):
    n, in_f = features.shape
    hdim = params['U_iou'].shape[0]
    out_f = params['cls_W'].shape[1]
    sizes = _level_sizes(n)

    b = 512
    grid = (pl.cdiv(n, b),)
    n_pad = grid[0] * b
    wspec = pl.BlockSpec(None, lambda i: (0, 0))

    def row2(v):
        return v.reshape(1, -1)

    xfiou = pl.pallas_call(
        _mlp_body,
        grid=grid,
        in_specs=[pl.BlockSpec((b, in_f), lambda i: (i, 0))] + [wspec] * 12,
        out_specs=pl.BlockSpec((b, 4 * hdim), lambda i: (i, 0)),
        out_shape=jax.ShapeDtypeStruct((n_pad, 4 * hdim), jnp.float32),
        compiler_params=pltpu.CompilerParams(
            dimension_semantics=("arbitrary",)),
    )(features,
      params['enc_W1'], row2(params['enc_b1']),
      params['enc_W2'], row2(params['enc_b2']),
      params['enc_W3'], row2(params['enc_b3']),
      params['enc_W4'], row2(params['enc_b4']),
      params['W_iou'], row2(params['b_iou']),
      params['W_f'], row2(params['b_f']))

    nd_last = sizes[-1]
    bufa_rows = ((nd_last + 2 * _CHUNK - 1) // (2 * _CHUNK)) * (2 * _CHUNK)
    bufb_rows = sizes[-2] if len(sizes) >= 2 else 8
    xs_rows = ((max(sizes) + 7) // 8) * 8

    sweep = partial(_sweep_body, sizes, hdim)
    hc = pl.pallas_call(
        sweep,
        in_specs=[
            pl.BlockSpec(memory_space=pltpu.MemorySpace.HBM),
            pl.BlockSpec(None, lambda: (0, 0)),
            pl.BlockSpec(None, lambda: (0, 0)),
        ],
        out_specs=pl.BlockSpec(memory_space=pltpu.MemorySpace.HBM),
        out_shape=jax.ShapeDtypeStruct((n_pad, 2 * hdim), jnp.float32),
        scratch_shapes=[
            pltpu.VMEM((bufa_rows, 2 * hdim), jnp.float32),
            pltpu.VMEM((bufb_rows, 2 * hdim), jnp.float32),
            pltpu.VMEM((xs_rows, 4 * hdim), jnp.float32),
            pltpu.SemaphoreType.DMA,
            pltpu.SemaphoreType.DMA,
        ],
        compiler_params=pltpu.CompilerParams(
            vmem_limit_bytes=100 * 1024 * 1024),
    )(xfiou, params['U_iou'], params['U_f'])

    out = pl.pallas_call(
        _cls_body,
        grid=grid,
        in_specs=[pl.BlockSpec((b, 2 * hdim), lambda i: (i, 0)),
                  wspec, wspec],
        out_specs=pl.BlockSpec((b, out_f), lambda i: (i, 0)),
        out_shape=jax.ShapeDtypeStruct((n, out_f), jnp.float32),
        compiler_params=pltpu.CompilerParams(
            dimension_semantics=("arbitrary",)),
    )(hc, params['cls_W'], row2(params['cls_b']))

    return out


# speedup vs baseline: 16.1310x; 1.8386x over previous
from functools import partial

import jax
import jax.numpy as jnp
from jax.experimental import pallas as pl
from jax.experimental.pallas import tpu as pltpu

_CHUNK = 2048


def _level_sizes(n):
    sizes = []
    d = 0
    while (1 << d) - 1 < n:
        s = (1 << d) - 1
        sizes.append(min((1 << (d + 1)) - 1, n) - s)
        d += 1
    return sizes


def _mlp_body(x_ref, w1_ref, b1_ref, w2_ref, b2_ref, w3_ref, b3_ref,
              w4_ref, b4_ref, wiou_ref, biou_ref, wf_ref, bf_ref, out_ref):
    def dot(a, b):
        return jax.lax.dot_general(a, b, (((1,), (0,)), ((), ())),
                                   preferred_element_type=jnp.float32)

    x = x_ref[...]
    for w_ref, b_ref in ((w1_ref, b1_ref), (w2_ref, b2_ref),
                         (w3_ref, b3_ref), (w4_ref, b4_ref)):
        x = jnp.maximum(dot(x, w_ref[...]) + b_ref[...], 0.0)
    xiou = dot(x, wiou_ref[...]) + biou_ref[...]
    xf = dot(x, wf_ref[...]) + bf_ref[...]
    out_ref[...] = jnp.concatenate([xiou, xf], axis=1)


def _sweep_body(sizes, hdim, xfiou_ref, uiou_ref, uf_ref, clsw_ref,
                clsb_ref, out_ref, xsa_ref, xsb_ref, bufa_ref, bufb_ref,
                oa_ref, ob_ref, sxa, sxb, soa, sob):
    uiou = uiou_ref[...]
    uf = uf_ref[...]
    clsw = clsw_ref[...]
    clsb = clsb_ref[...]
    nd_last = sizes[-1]
    za = (nd_last // 8) * 8
    bufa_rows = bufa_ref.shape[0]
    bufa_ref[za:bufa_rows, :] = jnp.zeros(
        (bufa_rows - za, bufa_ref.shape[1]), jnp.float32)

    xs_bufs = ((xsa_ref, sxa), (xsb_ref, sxb))
    o_bufs = ((oa_ref, soa), (ob_ref, sob))
    pending_o = [None, None]
    nlev = len(sizes)

    seg_fulls = []
    for step, d in enumerate(range(nlev - 1, -1, -1)):
        if d == nlev - 1:
            seg_fulls.append(0)
        elif d == nlev - 2:
            sf = min(bufa_rows // 2, sizes[d])
            sf = ((sf + _CHUNK - 1) // _CHUNK) * _CHUNK
            seg_fulls.append(min(sf, sizes[d]))
        else:
            seg_fulls.append(sizes[d])

    chunks = []
    for step, d in enumerate(range(nlev - 1, -1, -1)):
        n_d = sizes[d]
        for a in range(0, n_d, _CHUNK):
            chunks.append((step, d, a, min(a + _CHUNK, n_d)))

    def make_cp(k):
        step, d, a, b = chunks[k]
        s_g = (1 << d) - 1
        xr, sx = xs_bufs[k % 2]
        return pltpu.make_async_copy(
            xfiou_ref.at[pl.ds(s_g + a, b - a)], xr.at[pl.ds(0, b - a)], sx)

    cp = make_cp(0)
    cp.start()
    pending_x = cp
    for k, (step, d, a, b) in enumerate(chunks):
        s_g = (1 << d) - 1
        buf_ref = bufa_ref if step % 2 == 0 else bufb_ref
        obuf_ref = bufb_ref if step % 2 == 0 else bufa_ref
        xs_ref = xs_bufs[k % 2][0]
        pending_x.wait()
        if k + 1 < len(chunks):
            pending_x = make_cp(k + 1)
            pending_x.start()
        xio = xs_ref[0:b - a, :]
        xiou = xio[:, 0:3 * hdim]
        xf = xio[:, 3 * hdim:4 * hdim]
        if a < seg_fulls[step]:
            che = obuf_ref[2 * a:2 * b:2, :]
            cho = obuf_ref[2 * a + 1:2 * b:2, :]
            he = che[:, 0:hdim]
            ce = che[:, hdim:2 * hdim]
            ho = cho[:, 0:hdim]
            co = cho[:, hdim:2 * hdim]
            iou = xiou + jax.lax.dot_general(
                he + ho, uiou, (((1,), (0,)), ((), ())),
                preferred_element_type=jnp.float32)
            f_e = jax.nn.sigmoid(xf + jax.lax.dot_general(
                he, uf, (((1,), (0,)), ((), ())),
                preferred_element_type=jnp.float32))
            f_o = jax.nn.sigmoid(xf + jax.lax.dot_general(
                ho, uf, (((1,), (0,)), ((), ())),
                preferred_element_type=jnp.float32))
            csum = f_e * ce + f_o * co
        else:
            iou = xiou
            csum = None
        i_g = jax.nn.sigmoid(iou[:, 0:hdim])
        o_g = jax.nn.sigmoid(iou[:, hdim:2 * hdim])
        u_g = jnp.tanh(iou[:, 2 * hdim:3 * hdim])
        c_new = i_g * u_g
        if csum is not None:
            c_new = c_new + csum
        h_new = o_g * jnp.tanh(c_new)
        buf_ref[a:b, :] = jnp.concatenate([h_new, c_new], axis=1)
        logits = jax.lax.dot_general(
            h_new, clsw, (((1,), (0,)), ((), ())),
            preferred_element_type=jnp.float32) + clsb
        m = jnp.max(logits, axis=1, keepdims=True)
        ex = jnp.exp(logits - m)
        sm = ex / jnp.sum(ex, axis=1, keepdims=True)
        oi = k % 2
        orf, so = o_bufs[oi]
        if pending_o[oi] is not None:
            pending_o[oi].wait()
        orf[0:b - a, :] = sm
        cpo = pltpu.make_async_copy(
            orf.at[pl.ds(0, b - a)], out_ref.at[pl.ds(s_g + a, b - a)], so)
        cpo.start()
        pending_o[oi] = cpo
    for p in pending_o:
        if p is not None:
            p.wait()


def kernel(features, node_order, adjacency_list, edge_order, params):
    n, in_f = features.shape
    hdim = params['U_iou'].shape[0]
    out_f = params['cls_W'].shape[1]
    sizes = _level_sizes(n)

    b = 1024
    grid = (pl.cdiv(n, b),)
    n_pad = grid[0] * b
    wspec = pl.BlockSpec(None, lambda i: (0, 0))

    def row2(v):
        return v.reshape(1, -1)

    xfiou = pl.pallas_call(
        _mlp_body,
        grid=grid,
        in_specs=[pl.BlockSpec((b, in_f), lambda i: (i, 0))] + [wspec] * 12,
        out_specs=pl.BlockSpec((b, 4 * hdim), lambda i: (i, 0)),
        out_shape=jax.ShapeDtypeStruct((n_pad, 4 * hdim), jnp.float32),
        compiler_params=pltpu.CompilerParams(
            dimension_semantics=("arbitrary",)),
    )(features,
      params['enc_W1'], row2(params['enc_b1']),
      params['enc_W2'], row2(params['enc_b2']),
      params['enc_W3'], row2(params['enc_b3']),
      params['enc_W4'], row2(params['enc_b4']),
      params['W_iou'], row2(params['b_iou']),
      params['W_f'], row2(params['b_f']))

    nd_last = sizes[-1]
    bufa_rows = ((nd_last + 2 * _CHUNK - 1) // (2 * _CHUNK)) * (2 * _CHUNK)
    bufb_rows = sizes[-2] if len(sizes) >= 2 else 8

    sweep = partial(_sweep_body, sizes, hdim)
    out = pl.pallas_call(
        sweep,
        in_specs=[
            pl.BlockSpec(memory_space=pltpu.MemorySpace.HBM),
            pl.BlockSpec(None, lambda: (0, 0)),
            pl.BlockSpec(None, lambda: (0, 0)),
            pl.BlockSpec(None, lambda: (0, 0)),
            pl.BlockSpec(None, lambda: (0, 0)),
        ],
        out_specs=pl.BlockSpec(memory_space=pltpu.MemorySpace.HBM),
        out_shape=jax.ShapeDtypeStruct((n, out_f), jnp.float32),
        scratch_shapes=[
            pltpu.VMEM((_CHUNK, 4 * hdim), jnp.float32),
            pltpu.VMEM((_CHUNK, 4 * hdim), jnp.float32),
            pltpu.VMEM((bufa_rows, 2 * hdim), jnp.float32),
            pltpu.VMEM((bufb_rows, 2 * hdim), jnp.float32),
            pltpu.VMEM((_CHUNK, out_f), jnp.float32),
            pltpu.VMEM((_CHUNK, out_f), jnp.float32),
            pltpu.SemaphoreType.DMA,
            pltpu.SemaphoreType.DMA,
            pltpu.SemaphoreType.DMA,
            pltpu.SemaphoreType.DMA,
        ],
        compiler_params=pltpu.CompilerParams(
            vmem_limit_bytes=100 * 1024 * 1024),
    )(xfiou, params['U_iou'], params['U_f'], params['cls_W'],
      row2(params['cls_b']))

    return out


# speedup vs baseline: 22.5227x; 1.3962x over previous
from functools import partial

import jax
import jax.numpy as jnp
from jax.experimental import pallas as pl
from jax.experimental.pallas import tpu as pltpu

_CHUNK = 2048


def _level_sizes(n):
    sizes = []
    d = 0
    while (1 << d) - 1 < n:
        s = (1 << d) - 1
        sizes.append(min((1 << (d + 1)) - 1, n) - s)
        d += 1
    return sizes


def _dot(a, b):
    return jax.lax.dot_general(a, b, (((1,), (0,)), ((), ())),
                               preferred_element_type=jnp.float32)


def _sweep_body(sizes, hdim, feat_ref, w1_ref, b1_ref, w2_ref, b2_ref,
                w3_ref, b3_ref, w4_ref, b4_ref, wiou_ref, biou_ref,
                wf_ref, bf_ref, uiou_ref, uf2_ref, clsw_ref, clsb_ref,
                out_ref, xsa_ref, xsb_ref, bufa_ref, bufb_ref,
                oa_ref, ob_ref, sxa, sxb, soa, sob):
    uiou = uiou_ref[...]
    uf2 = uf2_ref[...]
    clsw = clsw_ref[...]
    clsb = clsb_ref[...]
    nd_last = sizes[-1]
    za = (nd_last // 8) * 8
    bufa_rows = bufa_ref.shape[0]
    bufa_ref[za:bufa_rows, :] = jnp.zeros(
        (bufa_rows - za, bufa_ref.shape[1]), jnp.float32)

    xs_bufs = ((xsa_ref, sxa), (xsb_ref, sxb))
    o_bufs = ((oa_ref, soa), (ob_ref, sob))
    pending_o = [None, None]
    nlev = len(sizes)

    seg_fulls = []
    for step, d in enumerate(range(nlev - 1, -1, -1)):
        if d == nlev - 1:
            seg_fulls.append(0)
        elif d == nlev - 2:
            sf = min(bufa_rows // 2, sizes[d])
            sf = ((sf + _CHUNK - 1) // _CHUNK) * _CHUNK
            seg_fulls.append(min(sf, sizes[d]))
        else:
            seg_fulls.append(sizes[d])

    chunks = []
    for step, d in enumerate(range(nlev - 1, -1, -1)):
        n_d = sizes[d]
        for a in range(0, n_d, _CHUNK):
            chunks.append((step, d, a, min(a + _CHUNK, n_d)))

    def make_cp(k):
        step, d, a, b = chunks[k]
        s_g = (1 << d) - 1
        xr, sx = xs_bufs[k % 2]
        return pltpu.make_async_copy(
            feat_ref.at[pl.ds(s_g + a, b - a)], xr.at[pl.ds(0, b - a)], sx)

    cp = make_cp(0)
    cp.start()
    pending_x = cp
    for k, (step, d, a, b) in enumerate(chunks):
        s_g = (1 << d) - 1
        buf_ref = bufa_ref if step % 2 == 0 else bufb_ref
        obuf_ref = bufb_ref if step % 2 == 0 else bufa_ref
        xs_ref = xs_bufs[k % 2][0]
        pending_x.wait()
        if k + 1 < len(chunks):
            pending_x = make_cp(k + 1)
            pending_x.start()
        x = xs_ref[0:b - a, :]
        for w_ref, b_ref in ((w1_ref, b1_ref), (w2_ref, b2_ref),
                             (w3_ref, b3_ref), (w4_ref, b4_ref)):
            x = jnp.maximum(_dot(x, w_ref[...]) + b_ref[...], 0.0)
        xiou = _dot(x, wiou_ref[...]) + biou_ref[...]
        xf = _dot(x, wf_ref[...]) + bf_ref[...]
        if a < seg_fulls[step]:
            che = obuf_ref[2 * a:2 * b:2, :]
            cho = obuf_ref[2 * a + 1:2 * b:2, :]
            he = che[:, 0:hdim]
            ce = che[:, hdim:2 * hdim]
            ho = cho[:, 0:hdim]
            co = cho[:, hdim:2 * hdim]
            iou = xiou + _dot(he + ho, uiou)
            f_eo = jax.nn.sigmoid(
                jnp.concatenate([xf, xf], axis=1)
                + _dot(jnp.concatenate([he, ho], axis=1), uf2))
            csum = (f_eo[:, 0:hdim] * ce + f_eo[:, hdim:2 * hdim] * co)
        else:
            iou = xiou
            csum = None
        i_g = jax.nn.sigmoid(iou[:, 0:hdim])
        o_g = jax.nn.sigmoid(iou[:, hdim:2 * hdim])
        u_g = jnp.tanh(iou[:, 2 * hdim:3 * hdim])
        c_new = i_g * u_g
        if csum is not None:
            c_new = c_new + csum
        h_new = o_g * jnp.tanh(c_new)
        buf_ref[a:b, :] = jnp.concatenate([h_new, c_new], axis=1)
        logits = _dot(h_new, clsw) + clsb
        m = jnp.max(logits, axis=1, keepdims=True)
        ex = jnp.exp(logits - m)
        sm = ex / jnp.sum(ex, axis=1, keepdims=True)
        oi = k % 2
        orf, so = o_bufs[oi]
        if pending_o[oi] is not None:
            pending_o[oi].wait()
        orf[0:b - a, :] = sm
        cpo = pltpu.make_async_copy(
            orf.at[pl.ds(0, b - a)], out_ref.at[pl.ds(s_g + a, b - a)], so)
        cpo.start()
        pending_o[oi] = cpo
    for p in pending_o:
        if p is not None:
            p.wait()


def kernel(features, node_order, adjacency_list, edge_order, params):
    n, in_f = features.shape
    hdim = params['U_iou'].shape[0]
    out_f = params['cls_W'].shape[1]
    sizes = _level_sizes(n)

    def row2(v):
        return v.reshape(1, -1)

    uf = params['U_f']
    zz = jnp.zeros((hdim, hdim), jnp.float32)
    uf2 = jnp.block([[uf, zz], [zz, uf]])

    nd_last = sizes[-1]
    bufa_rows = ((nd_last + 2 * _CHUNK - 1) // (2 * _CHUNK)) * (2 * _CHUNK)
    bufb_rows = sizes[-2] if len(sizes) >= 2 else 8

    wspec = pl.BlockSpec(None, lambda: (0, 0))
    sweep = partial(_sweep_body, sizes, hdim)
    out = pl.pallas_call(
        sweep,
        in_specs=[pl.BlockSpec(memory_space=pltpu.MemorySpace.HBM)]
        + [wspec] * 16,
        out_specs=pl.BlockSpec(memory_space=pltpu.MemorySpace.HBM),
        out_shape=jax.ShapeDtypeStruct((n, out_f), jnp.float32),
        scratch_shapes=[
            pltpu.VMEM((_CHUNK, in_f), jnp.float32),
            pltpu.VMEM((_CHUNK, in_f), jnp.float32),
            pltpu.VMEM((bufa_rows, 2 * hdim), jnp.float32),
            pltpu.VMEM((bufb_rows, 2 * hdim), jnp.float32),
            pltpu.VMEM((_CHUNK, out_f), jnp.float32),
            pltpu.VMEM((_CHUNK, out_f), jnp.float32),
            pltpu.SemaphoreType.DMA,
            pltpu.SemaphoreType.DMA,
            pltpu.SemaphoreType.DMA,
            pltpu.SemaphoreType.DMA,
        ],
        compiler_params=pltpu.CompilerParams(
            vmem_limit_bytes=100 * 1024 * 1024),
    )(features,
      params['enc_W1'], row2(params['enc_b1']),
      params['enc_W2'], row2(params['enc_b2']),
      params['enc_W3'], row2(params['enc_b3']),
      params['enc_W4'], row2(params['enc_b4']),
      params['W_iou'], row2(params['b_iou']),
      params['W_f'], row2(params['b_f']),
      params['U_iou'], uf2, params['cls_W'], row2(params['cls_b']))

    return out


# speedup vs baseline: 23.1245x; 1.0267x over previous
from functools import partial

import jax
import jax.numpy as jnp
from jax.experimental import pallas as pl
from jax.experimental.pallas import tpu as pltpu

_CHUNK = 4096


def _level_sizes(n):
    sizes = []
    d = 0
    while (1 << d) - 1 < n:
        s = (1 << d) - 1
        sizes.append(min((1 << (d + 1)) - 1, n) - s)
        d += 1
    return sizes


def _dot(a, b):
    return jax.lax.dot_general(a, b, (((1,), (0,)), ((), ())),
                               preferred_element_type=jnp.float32)


def _sweep_body(sizes, hdim, feat_ref, w1_ref, b1_ref, w2_ref, b2_ref,
                w3_ref, b3_ref, w4_ref, b4_ref, wiou_ref, biou_ref,
                wf_ref, bf_ref, uiou_ref, uf2_ref, clsw_ref, clsb_ref,
                out_ref, xsa_ref, xsb_ref, bufa_ref, bufb_ref,
                oa_ref, ob_ref, sxa, sxb, soa, sob):
    uiou = uiou_ref[...]
    uf2 = uf2_ref[...]
    clsw = clsw_ref[...]
    clsb = clsb_ref[...]
    nd_last = sizes[-1]
    za = (nd_last // 8) * 8
    bufa_rows = bufa_ref.shape[0]
    bufa_ref[za:bufa_rows, :] = jnp.zeros(
        (bufa_rows - za, bufa_ref.shape[1]), jnp.float32)

    xs_bufs = ((xsa_ref, sxa), (xsb_ref, sxb))
    o_bufs = ((oa_ref, soa), (ob_ref, sob))
    pending_o = [None, None]
    nlev = len(sizes)

    seg_fulls = []
    for step, d in enumerate(range(nlev - 1, -1, -1)):
        if d == nlev - 1:
            seg_fulls.append(0)
        elif d == nlev - 2:
            sf = min(bufa_rows // 2, sizes[d])
            sf = ((sf + _CHUNK - 1) // _CHUNK) * _CHUNK
            seg_fulls.append(min(sf, sizes[d]))
        else:
            seg_fulls.append(sizes[d])

    chunks = []
    for step, d in enumerate(range(nlev - 1, -1, -1)):
        n_d = sizes[d]
        for a in range(0, n_d, _CHUNK):
            chunks.append((step, d, a, min(a + _CHUNK, n_d)))

    def make_cp(k):
        step, d, a, b = chunks[k]
        s_g = (1 << d) - 1
        xr, sx = xs_bufs[k % 2]
        return pltpu.make_async_copy(
            feat_ref.at[pl.ds(s_g + a, b - a)], xr.at[pl.ds(0, b - a)], sx)

    cp = make_cp(0)
    cp.start()
    pending_x = cp
    for k, (step, d, a, b) in enumerate(chunks):
        s_g = (1 << d) - 1
        buf_ref = bufa_ref if step % 2 == 0 else bufb_ref
        obuf_ref = bufb_ref if step % 2 == 0 else bufa_ref
        xs_ref = xs_bufs[k % 2][0]
        pending_x.wait()
        if k + 1 < len(chunks):
            pending_x = make_cp(k + 1)
            pending_x.start()
        x = xs_ref[0:b - a, :]
        for w_ref, b_ref in ((w1_ref, b1_ref), (w2_ref, b2_ref),
                             (w3_ref, b3_ref), (w4_ref, b4_ref)):
            x = jnp.maximum(_dot(x, w_ref[...]) + b_ref[...], 0.0)
        xiou = _dot(x, wiou_ref[...]) + biou_ref[...]
        xf = _dot(x, wf_ref[...]) + bf_ref[...]
        if a < seg_fulls[step]:
            che = obuf_ref[2 * a:2 * b:2, :]
            cho = obuf_ref[2 * a + 1:2 * b:2, :]
            he = che[:, 0:hdim]
            ce = che[:, hdim:2 * hdim]
            ho = cho[:, 0:hdim]
            co = cho[:, hdim:2 * hdim]
            iou = xiou + _dot(he + ho, uiou)
            f_eo = jax.nn.sigmoid(
                jnp.concatenate([xf, xf], axis=1)
                + _dot(jnp.concatenate([he, ho], axis=1), uf2))
            csum = (f_eo[:, 0:hdim] * ce + f_eo[:, hdim:2 * hdim] * co)
        else:
            iou = xiou
            csum = None
        i_g = jax.nn.sigmoid(iou[:, 0:hdim])
        o_g = jax.nn.sigmoid(iou[:, hdim:2 * hdim])
        u_g = jnp.tanh(iou[:, 2 * hdim:3 * hdim])
        c_new = i_g * u_g
        if csum is not None:
            c_new = c_new + csum
        h_new = o_g * jnp.tanh(c_new)
        buf_ref[a:b, :] = jnp.concatenate([h_new, c_new], axis=1)
        logits = _dot(h_new, clsw) + clsb
        m = jnp.max(logits, axis=1, keepdims=True)
        ex = jnp.exp(logits - m)
        sm = ex / jnp.sum(ex, axis=1, keepdims=True)
        oi = k % 2
        orf, so = o_bufs[oi]
        if pending_o[oi] is not None:
            pending_o[oi].wait()
        orf[0:b - a, :] = sm
        cpo = pltpu.make_async_copy(
            orf.at[pl.ds(0, b - a)], out_ref.at[pl.ds(s_g + a, b - a)], so)
        cpo.start()
        pending_o[oi] = cpo
    for p in pending_o:
        if p is not None:
            p.wait()


def kernel(features, node_order, adjacency_list, edge_order, params):
    n, in_f = features.shape
    hdim = params['U_iou'].shape[0]
    out_f = params['cls_W'].shape[1]
    sizes = _level_sizes(n)

    def row2(v):
        return v.reshape(1, -1)

    uf = params['U_f']
    zz = jnp.zeros((hdim, hdim), jnp.float32)
    uf2 = jnp.block([[uf, zz], [zz, uf]])

    nd_last = sizes[-1]
    bufa_rows = ((nd_last + 2 * _CHUNK - 1) // (2 * _CHUNK)) * (2 * _CHUNK)
    bufb_rows = sizes[-2] if len(sizes) >= 2 else 8

    wspec = pl.BlockSpec(None, lambda: (0, 0))
    sweep = partial(_sweep_body, sizes, hdim)
    out = pl.pallas_call(
        sweep,
        in_specs=[pl.BlockSpec(memory_space=pltpu.MemorySpace.HBM)]
        + [wspec] * 16,
        out_specs=pl.BlockSpec(memory_space=pltpu.MemorySpace.HBM),
        out_shape=jax.ShapeDtypeStruct((n, out_f), jnp.float32),
        scratch_shapes=[
            pltpu.VMEM((_CHUNK, in_f), jnp.float32),
            pltpu.VMEM((_CHUNK, in_f), jnp.float32),
            pltpu.VMEM((bufa_rows, 2 * hdim), jnp.float32),
            pltpu.VMEM((bufb_rows, 2 * hdim), jnp.float32),
            pltpu.VMEM((_CHUNK, out_f), jnp.float32),
            pltpu.VMEM((_CHUNK, out_f), jnp.float32),
            pltpu.SemaphoreType.DMA,
            pltpu.SemaphoreType.DMA,
            pltpu.SemaphoreType.DMA,
            pltpu.SemaphoreType.DMA,
        ],
        compiler_params=pltpu.CompilerParams(
            vmem_limit_bytes=100 * 1024 * 1024),
    )(features,
      params['enc_W1'], row2(params['enc_b1']),
      params['enc_W2'], row2(params['enc_b2']),
      params['enc_W3'], row2(params['enc_b3']),
      params['enc_W4'], row2(params['enc_b4']),
      params['W_iou'], row2(params['b_iou']),
      params['W_f'], row2(params['b_f']),
      params['U_iou'], uf2, params['cls_W'], row2(params['cls_b']))

    return out


# speedup vs baseline: 23.5985x; 1.0205x over previous
from functools import partial

import jax
import jax.numpy as jnp
from jax.experimental import pallas as pl
from jax.experimental.pallas import tpu as pltpu

_CHUNK = 4096


def _level_sizes(n):
    sizes = []
    d = 0
    while (1 << d) - 1 < n:
        s = (1 << d) - 1
        sizes.append(min((1 << (d + 1)) - 1, n) - s)
        d += 1
    return sizes


def _dot(a, b):
    return jax.lax.dot_general(a, b, (((1,), (0,)), ((), ())),
                               preferred_element_type=jnp.float32)


def _sweep_body(sizes, hdim, feat_ref, w1_ref, b1_ref, w2_ref, b2_ref,
                w3_ref, b3_ref, w4_ref, b4_ref, wiou_ref, biou_ref,
                wf_ref, bf_ref, uiou_ref, uf2_ref, clsw_ref, clsb_ref,
                out_ref, xsa_ref, xsb_ref, xtop_ref, bufa_ref, bufb_ref,
                oa_ref, ob_ref, sxa, sxb, soa, sob):
    uiou = uiou_ref[...]
    uf2 = uf2_ref[...]
    clsw = clsw_ref[...]
    clsb = clsb_ref[...]
    nd_last = sizes[-1]
    za = (nd_last // 8) * 8
    bufa_rows = bufa_ref.shape[0]
    bufa_ref[za:bufa_rows, :] = jnp.zeros(
        (bufa_rows - za, bufa_ref.shape[1]), jnp.float32)

    xs_bufs = ((xsa_ref, sxa), (xsb_ref, sxb))
    o_bufs = ((oa_ref, soa), (ob_ref, sob))
    pending_o = [None, None]
    nlev = len(sizes)

    seg_fulls = []
    for step, d in enumerate(range(nlev - 1, -1, -1)):
        if d == nlev - 1:
            seg_fulls.append(0)
        elif d == nlev - 2:
            sf = min(bufa_rows // 2, sizes[d])
            sf = ((sf + _CHUNK - 1) // _CHUNK) * _CHUNK
            seg_fulls.append(min(sf, sizes[d]))
        else:
            seg_fulls.append(sizes[d])

    dtop = 0
    while dtop < nlev and (1 << (dtop + 1)) - 1 <= xtop_ref.shape[0]:
        dtop += 1
    top_rows = min((1 << dtop) - 1, sum(sizes))

    entries = []
    dma_list = []
    for step, d in enumerate(range(nlev - 1, -1, -1)):
        n_d = sizes[d]
        if d == dtop - 1:
            entries.append(('top_mlp', 0, 0, 0, top_rows))
            dma_list.append((0, top_rows))
        if d < dtop:
            entries.append(('top', step, d, 0, n_d))
        else:
            for a in range(0, n_d, _CHUNK):
                b = min(a + _CHUNK, n_d)
                entries.append(('deep', step, d, a, b))
                dma_list.append(((1 << d) - 1 + a, b - a))

    def make_cp(di):
        src, rows = dma_list[di]
        xr, sx = xs_bufs[di % 2]
        return pltpu.make_async_copy(
            feat_ref.at[pl.ds(src, rows)], xr.at[pl.ds(0, rows)], sx)

    def mlp(x):
        for w_ref, b_ref in ((w1_ref, b1_ref), (w2_ref, b2_ref),
                             (w3_ref, b3_ref), (w4_ref, b4_ref)):
            x = jnp.maximum(_dot(x, w_ref[...]) + b_ref[...], 0.0)
        xiou = _dot(x, wiou_ref[...]) + biou_ref[...]
        xf = _dot(x, wf_ref[...]) + bf_ref[...]
        return xiou, xf

    pending_x = make_cp(0)
    pending_x.start()
    di = 0
    state_o = [0]
    for kind, step, d, a, b in entries:
        if kind in ('deep', 'top_mlp'):
            xs_ref = xs_bufs[di % 2][0]
            pending_x.wait()
            if di + 1 < len(dma_list):
                pending_x = make_cp(di + 1)
                pending_x.start()
            di += 1
        if kind == 'top_mlp':
            xiou, xf = mlp(xs_ref[0:b - a, :])
            xtop_ref[0:b - a, :] = jnp.concatenate([xiou, xf], axis=1)
            continue
        s_g = (1 << d) - 1
        buf_ref = bufa_ref if step % 2 == 0 else bufb_ref
        obuf_ref = bufb_ref if step % 2 == 0 else bufa_ref
        if kind == 'deep':
            xiou, xf = mlp(xs_ref[0:b - a, :])
        else:
            xio = xtop_ref[s_g:s_g + b - a, :]
            xiou = xio[:, 0:3 * hdim]
            xf = xio[:, 3 * hdim:4 * hdim]
        if a < seg_fulls[step]:
            che = obuf_ref[2 * a:2 * b:2, :]
            cho = obuf_ref[2 * a + 1:2 * b:2, :]
            he = che[:, 0:hdim]
            ce = che[:, hdim:2 * hdim]
            ho = cho[:, 0:hdim]
            co = cho[:, hdim:2 * hdim]
            iou = xiou + _dot(he + ho, uiou)
            f_eo = jax.nn.sigmoid(
                jnp.concatenate([xf, xf], axis=1)
                + _dot(jnp.concatenate([he, ho], axis=1), uf2))
            csum = (f_eo[:, 0:hdim] * ce + f_eo[:, hdim:2 * hdim] * co)
        else:
            iou = xiou
            csum = None
        i_g = jax.nn.sigmoid(iou[:, 0:hdim])
        o_g = jax.nn.sigmoid(iou[:, hdim:2 * hdim])
        u_g = jnp.tanh(iou[:, 2 * hdim:3 * hdim])
        c_new = i_g * u_g
        if csum is not None:
            c_new = c_new + csum
        h_new = o_g * jnp.tanh(c_new)
        buf_ref[a:b, :] = jnp.concatenate([h_new, c_new], axis=1)
        logits = _dot(h_new, clsw) + clsb
        m = jnp.max(logits, axis=1, keepdims=True)
        ex = jnp.exp(logits - m)
        sm = ex / jnp.sum(ex, axis=1, keepdims=True)
        oi = state_o[0] % 2
        state_o[0] += 1
        orf, so = o_bufs[oi]
        if pending_o[oi] is not None:
            pending_o[oi].wait()
        orf[0:b - a, :] = sm
        cpo = pltpu.make_async_copy(
            orf.at[pl.ds(0, b - a)], out_ref.at[pl.ds(s_g + a, b - a)], so)
        cpo.start()
        pending_o[oi] = cpo
    for p in pending_o:
        if p is not None:
            p.wait()


def kernel(features, node_order, adjacency_list, edge_order, params):
    n, in_f = features.shape
    hdim = params['U_iou'].shape[0]
    out_f = params['cls_W'].shape[1]
    sizes = _level_sizes(n)

    def row2(v):
        return v.reshape(1, -1)

    uf = params['U_f']
    zz = jnp.zeros((hdim, hdim), jnp.float32)
    uf2 = jnp.block([[uf, zz], [zz, uf]])

    nd_last = sizes[-1]
    bufa_rows = ((nd_last + 2 * _CHUNK - 1) // (2 * _CHUNK)) * (2 * _CHUNK)
    bufb_rows = sizes[-2] if len(sizes) >= 2 else 8

    wspec = pl.BlockSpec(None, lambda: (0, 0))
    sweep = partial(_sweep_body, sizes, hdim)
    out = pl.pallas_call(
        sweep,
        in_specs=[pl.BlockSpec(memory_space=pltpu.MemorySpace.HBM)]
        + [wspec] * 16,
        out_specs=pl.BlockSpec(memory_space=pltpu.MemorySpace.HBM),
        out_shape=jax.ShapeDtypeStruct((n, out_f), jnp.float32),
        scratch_shapes=[
            pltpu.VMEM((_CHUNK, in_f), jnp.float32),
            pltpu.VMEM((_CHUNK, in_f), jnp.float32),
            pltpu.VMEM((_CHUNK, 4 * hdim), jnp.float32),
            pltpu.VMEM((bufa_rows, 2 * hdim), jnp.float32),
            pltpu.VMEM((bufb_rows, 2 * hdim), jnp.float32),
            pltpu.VMEM((_CHUNK, out_f), jnp.float32),
            pltpu.VMEM((_CHUNK, out_f), jnp.float32),
            pltpu.SemaphoreType.DMA,
            pltpu.SemaphoreType.DMA,
            pltpu.SemaphoreType.DMA,
            pltpu.SemaphoreType.DMA,
        ],
        compiler_params=pltpu.CompilerParams(
            vmem_limit_bytes=100 * 1024 * 1024),
    )(features,
      params['enc_W1'], row2(params['enc_b1']),
      params['enc_W2'], row2(params['enc_b2']),
      params['enc_W3'], row2(params['enc_b3']),
      params['enc_W4'], row2(params['enc_b4']),
      params['W_iou'], row2(params['b_iou']),
      params['W_f'], row2(params['b_f']),
      params['U_iou'], uf2, params['cls_W'], row2(params['cls_b']))

    return out
